# baseline (speedup 1.0000x reference)
import functools

import jax
import jax.numpy as jnp
from jax import lax
from jax.experimental import pallas as pl
from jax.experimental.pallas import tpu as pltpu
from jax.experimental.pallas import tpu_sc as plsc

_DESIRED_SIZE = 512.0
_EPS = 1e-7
_NUM_TILES = 16
_LANES = 16
_SLOT = 128


@functools.cache
def _build(n, n_conf_pad):
    assert n % _LANES == 0
    chunk = (n // (_NUM_TILES * 32)) * 32
    last = n - (_NUM_TILES - 1) * chunk
    assert last % 32 == 0 and last >= chunk
    steps = chunk * 4 // 64
    last_extra = (last - chunk) * 4 // 64
    conf_chunk = n_conf_pad // _NUM_TILES
    assert conf_chunk % 128 == 0
    conf_steps = conf_chunk // _LANES
    inv_n = 1.0 / n
    mesh = plsc.VectorSubcoreMesh(
        core_axis_name="c", subcore_axis_name="s", num_cores=1)

    @functools.partial(
        pl.kernel,
        mesh=mesh,
        out_type=jax.ShapeDtypeStruct((_SLOT,), jnp.float32),
        scratch_types=[
            pltpu.VMEM((last * 4 + 16,), jnp.float32),
            pltpu.VMEM((last * 4 + 16,), jnp.float32),
            pltpu.VMEM((conf_chunk,), jnp.float32),
            pltpu.VMEM((640,), jnp.float32),
            pltpu.VMEM((_SLOT,), jnp.float32),
            pltpu.VMEM_SHARED((_NUM_TILES * _SLOT,), jnp.float32),
            pltpu.VMEM((_NUM_TILES * _SLOT,), jnp.float32),
            pltpu.VMEM((_SLOT,), jnp.float32),
        ],
    )
    def sc_loss(pred_hbm, tgt_hbm, conf_hbm, out_hbm,
                pbuf, tbuf, cbuf, bounce, accv, shared, partials, outv):
        sid = lax.axis_index("s")
        is_last = sid == _NUM_TILES - 1
        base = sid * chunk * 4

        @pl.when(jnp.logical_not(is_last))
        def _():
            pltpu.sync_copy(pred_hbm.at[pl.ds(base, chunk * 4)],
                            pbuf.at[pl.ds(0, chunk * 4)])
            pltpu.sync_copy(tgt_hbm.at[pl.ds(base, chunk * 4)],
                            tbuf.at[pl.ds(0, chunk * 4)])

        @pl.when(is_last)
        def _():
            pltpu.sync_copy(pred_hbm.at[pl.ds(base, last * 4)],
                            pbuf.at[pl.ds(0, last * 4)])
            pltpu.sync_copy(tgt_hbm.at[pl.ds(base, last * 4)],
                            tbuf.at[pl.ds(0, last * 4)])

        pltpu.sync_copy(conf_hbm.at[pl.ds(sid * conf_chunk, conf_chunk)], cbuf)

        lane = lax.iota(jnp.int32, _LANES)
        group_mask = (lane & 3) == 0

        def pair(x, slot, combine):
            bounce[pl.ds(slot, _LANES)] = x
            return combine(x, bounce[pl.ds(slot + 1, _LANES)])

        def vec_term(off, j):
            s0 = j * 160
            pA = jnp.clip(pbuf[pl.ds(off, _LANES)], 0.0, _DESIRED_SIZE)
            pA2 = jnp.clip(pbuf[pl.ds(off + 2, _LANES)], 0.0, _DESIRED_SIZE)
            tA = jnp.clip(tbuf[pl.ds(off, _LANES)], 0.0, _DESIRED_SIZE)
            tA2 = jnp.clip(tbuf[pl.ds(off + 2, _LANES)], 0.0, _DESIRED_SIZE)
            plo = jnp.minimum(pA, pA2)
            phi = jnp.maximum(pA, pA2)
            phi = jnp.where(plo == phi, plo + 1.0, phi)
            tlo = jnp.minimum(tA, tA2)
            thi = jnp.maximum(tA, tA2)
            thi = jnp.where(tlo == thi, tlo + 1.0, thi)
            iwh = jnp.maximum(jnp.minimum(phi, thi) - jnp.maximum(plo, tlo),
                              0.0)
            pwh = phi - plo
            twh = thi - tlo
            cen = (plo + phi) - (tlo + thi)
            cen2 = cen * cen
            enc = jnp.maximum(phi, thi) - jnp.minimum(plo, tlo)
            enc2 = enc * enc
            mul = lambda a, b: a * b
            add = lambda a, b: a + b
            inter = pair(iwh, s0, mul)
            area_p = pair(pwh, s0 + 32, mul)
            area_t = pair(twh, s0 + 64, mul)
            rho2x4 = pair(cen2, s0 + 96, add)
            c2 = pair(enc2, s0 + 128, add) + _EPS
            union = area_p + area_t - inter + _EPS
            diou = inter / union - 0.25 * rho2x4 / c2
            return jnp.where(group_mask, 1.0 - diou, 0.0)

        def box_step(i, acc):
            o = i * 64
            t = vec_term(o, 0) + vec_term(o + 16, 1)
            t = t + vec_term(o + 32, 2) + vec_term(o + 48, 3)
            return acc + t

        acc = lax.fori_loop(0, steps, box_step,
                            jnp.zeros((_LANES,), jnp.float32))

        def conf_step(i, acc):
            c = cbuf[pl.ds(i * _LANES, _LANES)]
            return acc + (jnp.maximum(-c, 0.0)
                          + 0.5 * jnp.maximum(c - 1.0, 0.0))

        acc = lax.fori_loop(0, conf_steps, conf_step, acc)
        accv[pl.ds(0, _LANES)] = acc

        @pl.when(is_last)
        def _():
            extra = accv[pl.ds(0, _LANES)]
            for e in range(last_extra):
                extra = extra + box_step(steps + e,
                                         jnp.zeros((_LANES,), jnp.float32))
            accv[pl.ds(0, _LANES)] = extra

        pltpu.sync_copy(accv, shared.at[pl.ds(sid * _SLOT, _SLOT)])
        plsc.subcore_barrier()

        @pl.when(sid == 0)
        def _():
            pltpu.sync_copy(shared, partials)
            tot = jnp.zeros((_LANES,), jnp.float32)
            for j in range(_NUM_TILES):
                tot = tot + partials[pl.ds(j * _SLOT, _LANES)]
            total = jnp.float32(0.0)
            for j in range(_LANES):
                total = total + tot[j]
            outv[pl.ds(0, _LANES)] = jnp.full((_LANES,), total * inv_n,
                                              jnp.float32)
            pltpu.sync_copy(outv, out_hbm)

    return sc_loss


def kernel(pred_boxes, target_boxes, confidences):
    n = pred_boxes.shape[0]
    n_conf_pad = -(-n // (_NUM_TILES * 128)) * (_NUM_TILES * 128)
    conf = jnp.pad(confidences.reshape(n), (0, n_conf_pad - n))
    out = _build(n, n_conf_pad)(pred_boxes.reshape(n * 4),
                                target_boxes.reshape(n * 4),
                                conf)
    return out[0]
